# baseline (speedup 1.0000x reference)
import functools

import jax
import jax.numpy as jnp
from jax import lax
from jax.experimental import pallas as pl
from jax.experimental.pallas import tpu as pltpu
from jax.experimental.pallas import tpu_sc as plsc

N = 8
C = 96
GROUP = 8
EPS = 1e-05
L = 16
NCH = C // L


def _stats_body(x_ref, s_ref, q_ref):
    blk = x_ref[0]
    s_ref[...] = jnp.sum(blk, axis=(1, 2))[:, None, None]
    q_ref[...] = jnp.sum(blk * blk, axis=(1, 2))[:, None, None]


def _norm_body(x_ref, sc_ref, sh_ref, o_ref):
    o_ref[0] = x_ref[0] * sc_ref[...] + sh_ref[...]


def _sc_params_body(hw,
                    cs_hbm, cq_hbm, cst_hbm, cqt_hbm, fw_hbm, fb_hbm,
                    w_hbm, b_hbm,
                    scale_hbm, shift_hbm,
                    cs_v, cq_v, cst_v, cqt_v, fw_v, fb_v, w_v, b_v,
                    scale_v, shift_v):
    hwf = float(hw)
    cid = lax.axis_index("c")
    sid = lax.axis_index("s")

    @pl.when(jnp.logical_and(cid == 0, sid == 0))
    def _():
        pltpu.sync_copy(cs_hbm, cs_v)
        pltpu.sync_copy(cq_hbm, cq_v)
        pltpu.sync_copy(cst_hbm, cst_v)
        pltpu.sync_copy(cqt_hbm, cqt_v)
        pltpu.sync_copy(fw_hbm, fw_v)
        pltpu.sync_copy(fb_hbm, fb_v)
        pltpu.sync_copy(w_hbm, w_v)
        pltpu.sync_copy(b_hbm, b_v)

        fb_row = fb_v[pl.ds(0, L)]
        fw_rows = [fw_v[pl.ds(k * 2 * N, L)] for k in range(GROUP)]
        g_vecs = []
        for ch in range(NCH):
            means = []
            vars_ = []
            for n in range(N):
                cs_c = cs_v[pl.ds(n * C + ch * L, L)]
                cq_c = cq_v[pl.ds(n * C + ch * L, L)]
                m = cs_c * (1.0 / hwf)
                v = (cq_c - m * m * hwf) * (1.0 / (hwf - 1.0))
                means.append(m)
                vars_.append(v)
            best = None
            bidx = None
            for k in range(GROUP):
                row = fw_rows[k]
                acc = jnp.broadcast_to(fb_row[k], (L,))
                for n in range(N):
                    acc = acc + means[n] * row[n]
                    acc = acc + vars_[n] * row[N + n]
                if k == 0:
                    best = acc
                    bidx = jnp.zeros((L,), jnp.int32)
                else:
                    mgt = acc > best
                    best = jnp.where(mgt, acc, best)
                    bidx = jnp.where(mgt, jnp.full((L,), k, jnp.int32), bidx)
            g_vecs.append(bidx)

        g_sc = [g_vecs[c // L][c % L] for c in range(C)]

        zeros = jnp.zeros((L,), jnp.float32)
        gaccs = [zeros] * GROUP
        gaccq = [zeros] * GROUP
        for c in range(C):
            sv = cst_v[pl.ds(c * L, L)]
            qv = cqt_v[pl.ds(c * L, L)]
            gc = g_sc[c]
            for k in range(GROUP):
                hit = gc == k
                gaccs[k] = jnp.where(hit, gaccs[k] + sv, gaccs[k])
                gaccq[k] = jnp.where(hit, gaccq[k] + qv, gaccq[k])

        gm_list = []
        rstd_list = []
        for k in range(GROUP):
            gcount = jnp.broadcast_to(gaccs[k][N] * hwf, (L,))
            safe_n = jnp.maximum(gcount, 1.0)
            denom = jnp.maximum(gcount - 1.0, 1.0)
            gm = gaccs[k] / safe_n
            gv = (gaccq[k] - gcount * gm * gm) / denom
            a = gv + EPS
            i0 = lax.bitcast_convert_type(a, jnp.int32)
            i1 = jnp.full((L,), 0x5F3759DF, jnp.int32) - \
                lax.shift_right_arithmetic(i0, 1)
            y = lax.bitcast_convert_type(i1, jnp.float32)
            for _ in range(3):
                y = y * (1.5 - 0.5 * a * y * y)
            gm_list.append(gm)
            rstd_list.append(y)

        w_rows = [w_v[pl.ds(ch * L, L)] for ch in range(NCH)]
        b_rows = [b_v[pl.ds(ch * L, L)] for ch in range(NCH)]
        for c in range(C):
            gc = g_sc[c]
            mv = gm_list[0]
            rv = rstd_list[0]
            for k in range(1, GROUP):
                hit = gc == k
                mv = jnp.where(hit, gm_list[k], mv)
                rv = jnp.where(hit, rstd_list[k], rv)
            wc = w_rows[c // L][c % L]
            bc = b_rows[c // L][c % L]
            scl = wc * rv
            sft = bc - mv * scl
            scale_v[pl.ds(c * L, L)] = scl
            shift_v[pl.ds(c * L, L)] = sft

        pltpu.sync_copy(scale_v, scale_hbm)
        pltpu.sync_copy(shift_v, shift_hbm)


def _make_sc_params(hw):
    mesh = plsc.VectorSubcoreMesh(core_axis_name="c", subcore_axis_name="s")
    return pl.kernel(
        functools.partial(_sc_params_body, hw),
        mesh=mesh,
        out_type=[jax.ShapeDtypeStruct((C * L,), jnp.float32),
                  jax.ShapeDtypeStruct((C * L,), jnp.float32)],
        scratch_types=[
            pltpu.VMEM((N * C,), jnp.float32),
            pltpu.VMEM((N * C,), jnp.float32),
            pltpu.VMEM((C * L,), jnp.float32),
            pltpu.VMEM((C * L,), jnp.float32),
            pltpu.VMEM((GROUP * 2 * N,), jnp.float32),
            pltpu.VMEM((L,), jnp.float32),
            pltpu.VMEM((C,), jnp.float32),
            pltpu.VMEM((C,), jnp.float32),
            pltpu.VMEM((C * L,), jnp.float32),
            pltpu.VMEM((C * L,), jnp.float32),
        ],
    )


def kernel(x, fc_w, fc_b, weight, bias):
    n, c, H, W = x.shape
    HW = H * W
    R = n * c

    CB = 48
    NCB = c // CB

    s, q = pl.pallas_call(
        _stats_body,
        grid=(n, NCB),
        in_specs=[pl.BlockSpec((1, CB, H, W), lambda i, j: (i, j, 0, 0))],
        out_specs=[pl.BlockSpec((CB, 1, 1), lambda i, j: (i * NCB + j, 0, 0)),
                   pl.BlockSpec((CB, 1, 1), lambda i, j: (i * NCB + j, 0, 0))],
        out_shape=[jax.ShapeDtypeStruct((R, 1, 1), jnp.float32),
                   jax.ShapeDtypeStruct((R, 1, 1), jnp.float32)],
    )(x)

    cs = s.reshape(n, c)
    cq = q.reshape(n, c)
    cst = jnp.concatenate(
        [cs.T, jnp.ones((c, 1), jnp.float32),
         jnp.zeros((c, L - N - 1), jnp.float32)], axis=1).reshape(c * L)
    cqt = jnp.concatenate(
        [cq.T, jnp.zeros((c, L - N), jnp.float32)], axis=1).reshape(c * L)
    fw_flat = fc_w.reshape(GROUP * 2 * N)
    fb_pad = jnp.pad(fc_b, (0, L - GROUP))

    scale_t, shift_t = _make_sc_params(HW)(
        s.reshape(R), q.reshape(R), cst, cqt, fw_flat, fb_pad,
        weight.reshape(c), bias.reshape(c))

    scale = scale_t.reshape(c, L)[:, :n].T.reshape(R, 1, 1)
    shift = shift_t.reshape(c, L)[:, :n].T.reshape(R, 1, 1)

    out = pl.pallas_call(
        _norm_body,
        grid=(n, NCB),
        in_specs=[pl.BlockSpec((1, CB, H, W), lambda i, j: (i, j, 0, 0)),
                  pl.BlockSpec((CB, 1, 1), lambda i, j: (i * NCB + j, 0, 0)),
                  pl.BlockSpec((CB, 1, 1), lambda i, j: (i * NCB + j, 0, 0))],
        out_specs=pl.BlockSpec((1, CB, H, W), lambda i, j: (i, j, 0, 0)),
        out_shape=jax.ShapeDtypeStruct((n, c, H, W), jnp.float32),
    )(x, scale, shift)

    return out


# speedup vs baseline: 1.0254x; 1.0254x over previous
import functools

import jax
import jax.numpy as jnp
from jax import lax
from jax.experimental import pallas as pl
from jax.experimental.pallas import tpu as pltpu
from jax.experimental.pallas import tpu_sc as plsc

N = 8
C = 96
GROUP = 8
EPS = 1e-05
L = 16
NCH = C // L


def _stats_body(x_ref, s_ref, q_ref):
    blk = x_ref[0]
    s_ref[...] = jnp.sum(blk, axis=(1, 2))[:, None, None]
    q_ref[...] = jnp.sum(blk * blk, axis=(1, 2))[:, None, None]


def _norm_body(x_ref, sc_ref, sh_ref, o_ref):
    o_ref[0] = x_ref[0] * sc_ref[...] + sh_ref[...]


_OFF_FW = 0
_OFF_FB = _OFF_FW + GROUP * 2 * N
_OFF_W = _OFF_FB + L
_OFF_B = _OFF_W + C
_OFF_CST = _OFF_B + C
_OFF_CQT = _OFF_CST + C * L
_OFF_CS = _OFF_CQT + C * L
_OFF_CQ = _OFF_CS + N * C
_PACK = _OFF_CQ + N * C


def _sc_params_body(hw,
                    pack_hbm,
                    scale_hbm, shift_hbm,
                    pack_v, scale_v, shift_v):
    hwf = float(hw)
    cid = lax.axis_index("c")
    sid = lax.axis_index("s")

    @pl.when(jnp.logical_and(cid == 0, sid == 0))
    def _():
        pltpu.sync_copy(pack_hbm, pack_v)

        def cs_v(ix):
            return pack_v[pl.ds(_OFF_CS + ix, L)]

        def cq_v(ix):
            return pack_v[pl.ds(_OFF_CQ + ix, L)]

        def cst_v(ix):
            return pack_v[pl.ds(_OFF_CST + ix, L)]

        def cqt_v(ix):
            return pack_v[pl.ds(_OFF_CQT + ix, L)]

        fb_row = pack_v[pl.ds(_OFF_FB, L)]
        fw_rows = [pack_v[pl.ds(_OFF_FW + k * 2 * N, L)]
                   for k in range(GROUP)]
        g_vecs = []
        for ch in range(NCH):
            means = []
            vars_ = []
            for n in range(N):
                cs_c = cs_v(n * C + ch * L)
                cq_c = cq_v(n * C + ch * L)
                m = cs_c * (1.0 / hwf)
                v = (cq_c - m * m * hwf) * (1.0 / (hwf - 1.0))
                means.append(m)
                vars_.append(v)
            best = None
            bidx = None
            for k in range(GROUP):
                row = fw_rows[k]
                acc = jnp.broadcast_to(fb_row[k], (L,))
                for n in range(N):
                    acc = acc + means[n] * row[n]
                    acc = acc + vars_[n] * row[N + n]
                if k == 0:
                    best = acc
                    bidx = jnp.zeros((L,), jnp.int32)
                else:
                    mgt = acc > best
                    best = jnp.where(mgt, acc, best)
                    bidx = jnp.where(mgt, jnp.full((L,), k, jnp.int32), bidx)
            g_vecs.append(bidx)

        g_sc = [g_vecs[c // L][c % L] for c in range(C)]

        zeros = jnp.zeros((L,), jnp.float32)
        gaccs = [zeros] * GROUP
        gaccq = [zeros] * GROUP
        for c in range(C):
            sv = cst_v(c * L)
            qv = cqt_v(c * L)
            gc = g_sc[c]
            for k in range(GROUP):
                hit = gc == k
                gaccs[k] = jnp.where(hit, gaccs[k] + sv, gaccs[k])
                gaccq[k] = jnp.where(hit, gaccq[k] + qv, gaccq[k])

        gm_list = []
        rstd_list = []
        for k in range(GROUP):
            gcount = jnp.broadcast_to(gaccs[k][N] * hwf, (L,))
            safe_n = jnp.maximum(gcount, 1.0)
            denom = jnp.maximum(gcount - 1.0, 1.0)
            gm = gaccs[k] / safe_n
            gv = (gaccq[k] - gcount * gm * gm) / denom
            a = gv + EPS
            i0 = lax.bitcast_convert_type(a, jnp.int32)
            i1 = jnp.full((L,), 0x5F3759DF, jnp.int32) - \
                lax.shift_right_arithmetic(i0, 1)
            y = lax.bitcast_convert_type(i1, jnp.float32)
            for _ in range(3):
                y = y * (1.5 - 0.5 * a * y * y)
            gm_list.append(gm)
            rstd_list.append(y)

        w_rows = [pack_v[pl.ds(_OFF_W + ch * L, L)] for ch in range(NCH)]
        b_rows = [pack_v[pl.ds(_OFF_B + ch * L, L)] for ch in range(NCH)]
        for c in range(C):
            gc = g_sc[c]
            mv = gm_list[0]
            rv = rstd_list[0]
            for k in range(1, GROUP):
                hit = gc == k
                mv = jnp.where(hit, gm_list[k], mv)
                rv = jnp.where(hit, rstd_list[k], rv)
            wc = w_rows[c // L][c % L]
            bc = b_rows[c // L][c % L]
            scl = wc * rv
            sft = bc - mv * scl
            scale_v[pl.ds(c * L, L)] = scl
            shift_v[pl.ds(c * L, L)] = sft

        pltpu.sync_copy(scale_v, scale_hbm)
        pltpu.sync_copy(shift_v, shift_hbm)


def _make_sc_params(hw):
    mesh = plsc.VectorSubcoreMesh(core_axis_name="c", subcore_axis_name="s")
    return pl.kernel(
        functools.partial(_sc_params_body, hw),
        mesh=mesh,
        out_type=[jax.ShapeDtypeStruct((C * L,), jnp.float32),
                  jax.ShapeDtypeStruct((C * L,), jnp.float32)],
        scratch_types=[
            pltpu.VMEM((_PACK,), jnp.float32),
            pltpu.VMEM((C * L,), jnp.float32),
            pltpu.VMEM((C * L,), jnp.float32),
        ],
    )


def kernel(x, fc_w, fc_b, weight, bias):
    n, c, H, W = x.shape
    HW = H * W
    R = n * c

    CB = 48
    NCB = c // CB

    s, q = pl.pallas_call(
        _stats_body,
        grid=(n, NCB),
        in_specs=[pl.BlockSpec((1, CB, H, W), lambda i, j: (i, j, 0, 0))],
        out_specs=[pl.BlockSpec((CB, 1, 1), lambda i, j: (i * NCB + j, 0, 0)),
                   pl.BlockSpec((CB, 1, 1), lambda i, j: (i * NCB + j, 0, 0))],
        out_shape=[jax.ShapeDtypeStruct((R, 1, 1), jnp.float32),
                   jax.ShapeDtypeStruct((R, 1, 1), jnp.float32)],
    )(x)

    cs = s.reshape(n, c)
    cq = q.reshape(n, c)
    cst = jnp.concatenate(
        [cs.T, jnp.ones((c, 1), jnp.float32),
         jnp.zeros((c, L - N - 1), jnp.float32)], axis=1).reshape(c * L)
    cqt = jnp.concatenate(
        [cq.T, jnp.zeros((c, L - N), jnp.float32)], axis=1).reshape(c * L)
    pack = jnp.concatenate([
        fc_w.reshape(GROUP * 2 * N),
        jnp.pad(fc_b, (0, L - GROUP)),
        weight.reshape(c),
        bias.reshape(c),
        cst,
        cqt,
        s.reshape(R),
        q.reshape(R),
    ])

    scale_t, shift_t = _make_sc_params(HW)(pack)

    scale = scale_t.reshape(c, L)[:, :n].T.reshape(R, 1, 1)
    shift = shift_t.reshape(c, L)[:, :n].T.reshape(R, 1, 1)

    out = pl.pallas_call(
        _norm_body,
        grid=(n, NCB),
        in_specs=[pl.BlockSpec((1, CB, H, W), lambda i, j: (i, j, 0, 0)),
                  pl.BlockSpec((CB, 1, 1), lambda i, j: (i * NCB + j, 0, 0)),
                  pl.BlockSpec((CB, 1, 1), lambda i, j: (i * NCB + j, 0, 0))],
        out_specs=pl.BlockSpec((1, CB, H, W), lambda i, j: (i, j, 0, 0)),
        out_shape=jax.ShapeDtypeStruct((n, c, H, W), jnp.float32),
    )(x, scale, shift)

    return out


# speedup vs baseline: 1.0337x; 1.0080x over previous
import functools

import jax
import jax.numpy as jnp
from jax import lax
from jax.experimental import pallas as pl
from jax.experimental.pallas import tpu as pltpu
from jax.experimental.pallas import tpu_sc as plsc

N = 8
C = 96
GROUP = 8
EPS = 1e-05
L = 16
NCH = C // L


def _stats_body(x_ref, s_ref, q_ref):
    blk = x_ref[0]
    s_ref[...] = jnp.sum(blk, axis=(1, 2))[:, None, None]
    q_ref[...] = jnp.sum(blk * blk, axis=(1, 2))[:, None, None]


def _norm_body(x_ref, sc_ref, sh_ref, o_ref):
    o_ref[0] = x_ref[0] * sc_ref[...] + sh_ref[...]


_OFF_FW = 0
_OFF_FB = _OFF_FW + GROUP * 2 * N
_OFF_W = _OFF_FB + L
_OFF_B = _OFF_W + C
_OFF_CST = _OFF_B + C
_OFF_CQT = _OFF_CST + C * L
_OFF_CS = _OFF_CQT + C * L
_OFF_CQ = _OFF_CS + N * C
_PACK = _OFF_CQ + N * C


def _sc_params_body(hw,
                    pack_hbm,
                    scale_hbm, shift_hbm,
                    pack_v, scale_v, shift_v):
    hwf = float(hw)
    cid = lax.axis_index("c")
    sid = lax.axis_index("s")

    @pl.when(jnp.logical_and(cid == 0, sid == 0))
    def _():
        pltpu.sync_copy(pack_hbm, pack_v)

        def cs_v(ix):
            return pack_v[pl.ds(_OFF_CS + ix, L)]

        def cq_v(ix):
            return pack_v[pl.ds(_OFF_CQ + ix, L)]

        def cst_v(ix):
            return pack_v[pl.ds(_OFF_CST + ix, L)]

        def cqt_v(ix):
            return pack_v[pl.ds(_OFF_CQT + ix, L)]

        fb_row = pack_v[pl.ds(_OFF_FB, L)]
        fw_rows = [pack_v[pl.ds(_OFF_FW + k * 2 * N, L)]
                   for k in range(GROUP)]
        g_vecs = []
        for ch in range(NCH):
            means = []
            vars_ = []
            for n in range(N):
                cs_c = cs_v(n * C + ch * L)
                cq_c = cq_v(n * C + ch * L)
                m = cs_c * (1.0 / hwf)
                v = (cq_c - m * m * hwf) * (1.0 / (hwf - 1.0))
                means.append(m)
                vars_.append(v)
            best = None
            bidx = None
            for k in range(GROUP):
                row = fw_rows[k]
                acc = jnp.broadcast_to(fb_row[k], (L,))
                for n in range(N):
                    acc = acc + means[n] * row[n]
                    acc = acc + vars_[n] * row[N + n]
                if k == 0:
                    best = acc
                    bidx = jnp.zeros((L,), jnp.int32)
                else:
                    mgt = acc > best
                    best = jnp.where(mgt, acc, best)
                    bidx = jnp.where(mgt, jnp.full((L,), k, jnp.int32), bidx)
            g_vecs.append(bidx)

        g_sc = [g_vecs[c // L][c % L] for c in range(C)]

        zeros = jnp.zeros((L,), jnp.float32)
        gaccs = [zeros] * GROUP
        gaccq = [zeros] * GROUP
        for c in range(C):
            sv = cst_v(c * L)
            qv = cqt_v(c * L)
            gc = g_sc[c]
            for k in range(GROUP):
                hit = gc == k
                gaccs[k] = jnp.where(hit, gaccs[k] + sv, gaccs[k])
                gaccq[k] = jnp.where(hit, gaccq[k] + qv, gaccq[k])

        gm_list = []
        rstd_list = []
        for k in range(GROUP):
            gcount = jnp.broadcast_to(gaccs[k][N] * hwf, (L,))
            safe_n = jnp.maximum(gcount, 1.0)
            denom = jnp.maximum(gcount - 1.0, 1.0)
            gm = gaccs[k] / safe_n
            gv = (gaccq[k] - gcount * gm * gm) / denom
            a = gv + EPS
            i0 = lax.bitcast_convert_type(a, jnp.int32)
            i1 = jnp.full((L,), 0x5F3759DF, jnp.int32) - \
                lax.shift_right_arithmetic(i0, 1)
            y = lax.bitcast_convert_type(i1, jnp.float32)
            for _ in range(3):
                y = y * (1.5 - 0.5 * a * y * y)
            gm_list.append(gm)
            rstd_list.append(y)

        w_rows = [pack_v[pl.ds(_OFF_W + ch * L, L)] for ch in range(NCH)]
        b_rows = [pack_v[pl.ds(_OFF_B + ch * L, L)] for ch in range(NCH)]
        for c in range(C):
            gc = g_sc[c]
            mv = gm_list[0]
            rv = rstd_list[0]
            for k in range(1, GROUP):
                hit = gc == k
                mv = jnp.where(hit, gm_list[k], mv)
                rv = jnp.where(hit, rstd_list[k], rv)
            wc = w_rows[c // L][c % L]
            bc = b_rows[c // L][c % L]
            scl = wc * rv
            sft = bc - mv * scl
            scale_v[pl.ds(c * L, L)] = scl
            shift_v[pl.ds(c * L, L)] = sft

        pltpu.sync_copy(scale_v, scale_hbm)
        pltpu.sync_copy(shift_v, shift_hbm)


def _make_sc_params(hw):
    mesh = plsc.VectorSubcoreMesh(core_axis_name="c", subcore_axis_name="s",
                                  num_cores=1)
    return pl.kernel(
        functools.partial(_sc_params_body, hw),
        mesh=mesh,
        out_type=[jax.ShapeDtypeStruct((C * L,), jnp.float32),
                  jax.ShapeDtypeStruct((C * L,), jnp.float32)],
        scratch_types=[
            pltpu.VMEM((_PACK,), jnp.float32),
            pltpu.VMEM((C * L,), jnp.float32),
            pltpu.VMEM((C * L,), jnp.float32),
        ],
    )


def kernel(x, fc_w, fc_b, weight, bias):
    n, c, H, W = x.shape
    HW = H * W
    R = n * c

    CB = 48
    NCB = c // CB

    s, q = pl.pallas_call(
        _stats_body,
        grid=(n, NCB),
        in_specs=[pl.BlockSpec((1, CB, H, W), lambda i, j: (i, j, 0, 0))],
        out_specs=[pl.BlockSpec((CB, 1, 1), lambda i, j: (i * NCB + j, 0, 0)),
                   pl.BlockSpec((CB, 1, 1), lambda i, j: (i * NCB + j, 0, 0))],
        out_shape=[jax.ShapeDtypeStruct((R, 1, 1), jnp.float32),
                   jax.ShapeDtypeStruct((R, 1, 1), jnp.float32)],
    )(x)

    cs = s.reshape(n, c)
    cq = q.reshape(n, c)
    cst = jnp.concatenate(
        [cs.T, jnp.ones((c, 1), jnp.float32),
         jnp.zeros((c, L - N - 1), jnp.float32)], axis=1).reshape(c * L)
    cqt = jnp.concatenate(
        [cq.T, jnp.zeros((c, L - N), jnp.float32)], axis=1).reshape(c * L)
    pack = jnp.concatenate([
        fc_w.reshape(GROUP * 2 * N),
        jnp.pad(fc_b, (0, L - GROUP)),
        weight.reshape(c),
        bias.reshape(c),
        cst,
        cqt,
        s.reshape(R),
        q.reshape(R),
    ])

    scale_t, shift_t = _make_sc_params(HW)(pack)

    scale = scale_t.reshape(c, L)[:, :n].T.reshape(R, 1, 1)
    shift = shift_t.reshape(c, L)[:, :n].T.reshape(R, 1, 1)

    out = pl.pallas_call(
        _norm_body,
        grid=(n, NCB),
        in_specs=[pl.BlockSpec((1, CB, H, W), lambda i, j: (i, j, 0, 0)),
                  pl.BlockSpec((CB, 1, 1), lambda i, j: (i * NCB + j, 0, 0)),
                  pl.BlockSpec((CB, 1, 1), lambda i, j: (i * NCB + j, 0, 0))],
        out_specs=pl.BlockSpec((1, CB, H, W), lambda i, j: (i, j, 0, 0)),
        out_shape=jax.ShapeDtypeStruct((n, c, H, W), jnp.float32),
    )(x, scale, shift)

    return out


# speedup vs baseline: 1.0515x; 1.0173x over previous
import functools

import jax
import jax.numpy as jnp
from jax import lax
from jax.experimental import pallas as pl
from jax.experimental.pallas import tpu as pltpu
from jax.experimental.pallas import tpu_sc as plsc

N = 8
C = 96
GROUP = 8
EPS = 1e-05
L = 16
NCH = C // L


def _stats_body(x_ref, s_ref, q_ref):
    blk = x_ref[0]
    s_ref[...] = jnp.sum(blk, axis=(1, 2))[:, None]
    q_ref[...] = jnp.sum(blk * blk, axis=(1, 2))[:, None]


def _norm_body(x_ref, sc_ref, sh_ref, o_ref):
    o_ref[0] = x_ref[0] * sc_ref[...][:, :, None] + sh_ref[...][:, :, None]


_OFF_FW = 0
_OFF_FB = _OFF_FW + GROUP * 2 * N
_OFF_W = _OFF_FB + L
_OFF_B = _OFF_W + C
_OFF_CST = _OFF_B + C
_OFF_CQT = _OFF_CST + C * L
_OFF_CS = _OFF_CQT + C * L
_OFF_CQ = _OFF_CS + N * C
_PACK = _OFF_CQ + N * C


def _sc_params_body(hw,
                    pack_hbm,
                    scale_hbm, shift_hbm,
                    pack_v, scale_v, shift_v):
    hwf = float(hw)
    cid = lax.axis_index("c")
    sid = lax.axis_index("s")

    @pl.when(jnp.logical_and(cid == 0, sid == 0))
    def _():
        pltpu.sync_copy(pack_hbm, pack_v)

        def cs_v(ix):
            return pack_v[pl.ds(_OFF_CS + ix, L)]

        def cq_v(ix):
            return pack_v[pl.ds(_OFF_CQ + ix, L)]

        def cst_v(ix):
            return pack_v[pl.ds(_OFF_CST + ix, L)]

        def cqt_v(ix):
            return pack_v[pl.ds(_OFF_CQT + ix, L)]

        fb_row = pack_v[pl.ds(_OFF_FB, L)]
        fw_rows = [pack_v[pl.ds(_OFF_FW + k * 2 * N, L)]
                   for k in range(GROUP)]
        g_vecs = []
        for ch in range(NCH):
            means = []
            vars_ = []
            for n in range(N):
                cs_c = cs_v(n * C + ch * L)
                cq_c = cq_v(n * C + ch * L)
                m = cs_c * (1.0 / hwf)
                v = (cq_c - m * m * hwf) * (1.0 / (hwf - 1.0))
                means.append(m)
                vars_.append(v)
            best = None
            bidx = None
            for k in range(GROUP):
                row = fw_rows[k]
                acc = jnp.broadcast_to(fb_row[k], (L,))
                for n in range(N):
                    acc = acc + means[n] * row[n]
                    acc = acc + vars_[n] * row[N + n]
                if k == 0:
                    best = acc
                    bidx = jnp.zeros((L,), jnp.int32)
                else:
                    mgt = acc > best
                    best = jnp.where(mgt, acc, best)
                    bidx = jnp.where(mgt, jnp.full((L,), k, jnp.int32), bidx)
            g_vecs.append(bidx)

        g_sc = [g_vecs[c // L][c % L] for c in range(C)]

        zeros = jnp.zeros((L,), jnp.float32)
        gaccs = [zeros] * GROUP
        gaccq = [zeros] * GROUP
        for c in range(C):
            sv = cst_v(c * L)
            qv = cqt_v(c * L)
            gc = g_sc[c]
            for k in range(GROUP):
                hit = gc == k
                gaccs[k] = jnp.where(hit, gaccs[k] + sv, gaccs[k])
                gaccq[k] = jnp.where(hit, gaccq[k] + qv, gaccq[k])

        gm_list = []
        rstd_list = []
        for k in range(GROUP):
            gcount = jnp.broadcast_to(gaccs[k][N] * hwf, (L,))
            safe_n = jnp.maximum(gcount, 1.0)
            denom = jnp.maximum(gcount - 1.0, 1.0)
            gm = gaccs[k] / safe_n
            gv = (gaccq[k] - gcount * gm * gm) / denom
            a = gv + EPS
            i0 = lax.bitcast_convert_type(a, jnp.int32)
            i1 = jnp.full((L,), 0x5F3759DF, jnp.int32) - \
                lax.shift_right_arithmetic(i0, 1)
            y = lax.bitcast_convert_type(i1, jnp.float32)
            for _ in range(3):
                y = y * (1.5 - 0.5 * a * y * y)
            gm_list.append(gm)
            rstd_list.append(y)

        w_rows = [pack_v[pl.ds(_OFF_W + ch * L, L)] for ch in range(NCH)]
        b_rows = [pack_v[pl.ds(_OFF_B + ch * L, L)] for ch in range(NCH)]
        for c in range(C):
            gc = g_sc[c]
            mv = gm_list[0]
            rv = rstd_list[0]
            for k in range(1, GROUP):
                hit = gc == k
                mv = jnp.where(hit, gm_list[k], mv)
                rv = jnp.where(hit, rstd_list[k], rv)
            wc = w_rows[c // L][c % L]
            bc = b_rows[c // L][c % L]
            scl = wc * rv
            sft = bc - mv * scl
            scale_v[pl.ds(c * L, L)] = scl
            shift_v[pl.ds(c * L, L)] = sft

        pltpu.sync_copy(scale_v, scale_hbm)
        pltpu.sync_copy(shift_v, shift_hbm)


def _make_sc_params(hw):
    mesh = plsc.VectorSubcoreMesh(core_axis_name="c", subcore_axis_name="s",
                                  num_cores=1)
    return pl.kernel(
        functools.partial(_sc_params_body, hw),
        mesh=mesh,
        out_type=[jax.ShapeDtypeStruct((C * L,), jnp.float32),
                  jax.ShapeDtypeStruct((C * L,), jnp.float32)],
        scratch_types=[
            pltpu.VMEM((_PACK,), jnp.float32),
            pltpu.VMEM((C * L,), jnp.float32),
            pltpu.VMEM((C * L,), jnp.float32),
        ],
    )


def kernel(x, fc_w, fc_b, weight, bias):
    n, c, H, W = x.shape
    HW = H * W
    R = n * c

    CB = 48
    NCB = c // CB

    s, q = pl.pallas_call(
        _stats_body,
        grid=(n, NCB),
        in_specs=[pl.BlockSpec((1, CB, H, W), lambda i, j: (i, j, 0, 0))],
        out_specs=[pl.BlockSpec((CB, 1), lambda i, j: (i * NCB + j, 0)),
                   pl.BlockSpec((CB, 1), lambda i, j: (i * NCB + j, 0))],
        out_shape=[jax.ShapeDtypeStruct((R, 1), jnp.float32),
                   jax.ShapeDtypeStruct((R, 1), jnp.float32)],
    )(x)

    cs = s.reshape(n, c)
    cq = q.reshape(n, c)
    cst = jnp.concatenate(
        [cs.T, jnp.ones((c, 1), jnp.float32),
         jnp.zeros((c, L - N - 1), jnp.float32)], axis=1).reshape(c * L)
    cqt = jnp.concatenate(
        [cq.T, jnp.zeros((c, L - N), jnp.float32)], axis=1).reshape(c * L)
    pack = jnp.concatenate([
        fc_w.reshape(GROUP * 2 * N),
        jnp.pad(fc_b, (0, L - GROUP)),
        weight.reshape(c),
        bias.reshape(c),
        cst,
        cqt,
        s.reshape(R),
        q.reshape(R),
    ])

    scale_t, shift_t = _make_sc_params(HW)(pack)

    scale = scale_t.reshape(c, L)[:, :n].T.reshape(R, 1)
    shift = shift_t.reshape(c, L)[:, :n].T.reshape(R, 1)

    out = pl.pallas_call(
        _norm_body,
        grid=(n, NCB),
        in_specs=[pl.BlockSpec((1, CB, H, W), lambda i, j: (i, j, 0, 0)),
                  pl.BlockSpec((CB, 1), lambda i, j: (i * NCB + j, 0)),
                  pl.BlockSpec((CB, 1), lambda i, j: (i * NCB + j, 0))],
        out_specs=pl.BlockSpec((1, CB, H, W), lambda i, j: (i, j, 0, 0)),
        out_shape=jax.ShapeDtypeStruct((n, c, H, W), jnp.float32),
    )(x, scale, shift)

    return out
